# baseline (speedup 1.0000x reference)
import jax
import jax.numpy as jnp
from jax.experimental import pallas as pl


def _vq_body(x_ref, cb_ref, out_ref):
    x = x_ref[...]
    cb = cb_ref[...]
    xn = jnp.sum(x * x, axis=1, keepdims=True)
    cn = jnp.sum(cb * cb, axis=1)[None, :]
    s = jax.lax.dot_general(
        x, cb, (((1,), (1,)), ((), ())),
        preferred_element_type=jnp.float32)
    dist = xn - 2.0 * s + cn
    m = jnp.min(dist, axis=1, keepdims=True)
    k = dist.shape[1]
    iota = jax.lax.broadcasted_iota(jnp.int32, dist.shape, 1)
    idx = jnp.min(jnp.where(dist == m, iota, k), axis=1).astype(jnp.int32)
    out_ref[...] = idx.reshape(out_ref.shape)


def kernel(z_e_x, codebook):
    B, D, H, W = z_e_x.shape
    K = codebook.shape[0]
    flat = jnp.transpose(z_e_x, (0, 2, 3, 1)).reshape(-1, D)
    N = flat.shape[0]
    BLK = 4096
    out = pl.pallas_call(
        _vq_body,
        grid=(N // BLK,),
        in_specs=[
            pl.BlockSpec((BLK, D), lambda i: (i, 0)),
            pl.BlockSpec((K, D), lambda i: (0, 0)),
        ],
        out_specs=pl.BlockSpec((1, 1, BLK), lambda i: (i, 0, 0)),
        out_shape=jax.ShapeDtypeStruct((N // BLK, 1, BLK), jnp.int32),
    )(flat, codebook)
    return out.reshape(B, H, W)


# speedup vs baseline: 1.0705x; 1.0705x over previous
import jax
import jax.numpy as jnp
from jax.experimental import pallas as pl


def _vq_body(x_ref, cb_ref, out_ref):
    x = x_ref[0]
    cb = cb_ref[...]
    k_count, _ = cb.shape
    cb2 = -2.0 * cb
    xn = jnp.sum(x * x, axis=0, keepdims=True)
    cn = jnp.sum(cb * cb, axis=1, keepdims=True)
    s2 = jax.lax.dot_general(
        cb2, x, (((1,), (0,)), ((), ())),
        preferred_element_type=jnp.float32)
    dist = (xn + s2) + cn
    m = jnp.min(dist, axis=0, keepdims=True)
    eqf = jnp.where(dist == m, 1.0, 0.0)
    kf = jax.lax.broadcasted_iota(
        jnp.int32, (1, k_count), 1).astype(jnp.float32)
    w = jnp.concatenate([jnp.ones_like(kf), kf, kf * kf], axis=0)
    r = jax.lax.dot_general(
        w, eqf, (((1,), (0,)), ((), ())),
        preferred_element_type=jnp.float32,
        precision=jax.lax.Precision.HIGHEST)
    cnt, s, q = r[0:1], r[1:2], r[2:3]
    tie = (s - jnp.sqrt(jnp.maximum(2.0 * q - s * s, 0.0))) * 0.5
    idx_f = jnp.where(cnt > 1.0, tie, s)
    out_ref[...] = (idx_f + 0.5).astype(jnp.int32).reshape(out_ref.shape)


def kernel(z_e_x, codebook):
    B, D, H, W = z_e_x.shape
    K = codebook.shape[0]
    N = H * W
    z = z_e_x.reshape(B, D, N)
    out = pl.pallas_call(
        _vq_body,
        grid=(B,),
        in_specs=[
            pl.BlockSpec((1, D, N), lambda i: (i, 0, 0)),
            pl.BlockSpec((K, D), lambda i: (0, 0)),
        ],
        out_specs=pl.BlockSpec((1, 1, N), lambda i: (i, 0, 0)),
        out_shape=jax.ShapeDtypeStruct((B, 1, N), jnp.int32),
    )(z, codebook)
    return out.reshape(B, H, W)


# speedup vs baseline: 1.9090x; 1.7832x over previous
import jax
import jax.numpy as jnp
from jax.experimental import pallas as pl


def _vq_body(x_ref, cb_ref, out_ref):
    x = x_ref[0]
    cb = cb_ref[...]
    k_count, _ = cb.shape
    cb2 = -2.0 * cb
    xn = jnp.sum(x * x, axis=0, keepdims=True)
    cn = jnp.sum(cb * cb, axis=1, keepdims=True)
    s2 = jax.lax.dot_general(
        cb2, x, (((1,), (0,)), ((), ())),
        preferred_element_type=jnp.float32)
    dist = (xn + s2) + cn
    m = jnp.min(dist, axis=0, keepdims=True)
    kio = jax.lax.broadcasted_iota(jnp.int32, dist.shape, 0)
    idx = jnp.min(jnp.where(dist == m, kio, k_count), axis=0)
    out_ref[...] = idx.reshape(out_ref.shape)


def kernel(z_e_x, codebook):
    B, D, H, W = z_e_x.shape
    K = codebook.shape[0]
    N = H * W
    z = z_e_x.reshape(B, D, N)
    out = pl.pallas_call(
        _vq_body,
        grid=(B,),
        in_specs=[
            pl.BlockSpec((1, D, N), lambda i: (i, 0, 0)),
            pl.BlockSpec((K, D), lambda i: (0, 0)),
        ],
        out_specs=pl.BlockSpec((1, 1, N), lambda i: (i, 0, 0)),
        out_shape=jax.ShapeDtypeStruct((B, 1, N), jnp.int32),
    )(z, codebook)
    return out.reshape(B, H, W)


# speedup vs baseline: 1.9871x; 1.0409x over previous
import jax
import jax.numpy as jnp
from jax.experimental import pallas as pl


def _vq_body(x_ref, cb_ref, out_ref):
    imgs = x_ref.shape[0]
    cb = cb_ref[...]
    k_count, _ = cb.shape
    cb2 = -2.0 * cb
    cn = jnp.sum(cb * cb, axis=1, keepdims=True)
    kio = jax.lax.broadcasted_iota(
        jnp.int32, (k_count, x_ref.shape[2]), 0)
    for j in range(imgs):
        x = x_ref[j]
        xn = jnp.sum(x * x, axis=0, keepdims=True)
        s2 = jax.lax.dot_general(
            cb2, x, (((1,), (0,)), ((), ())),
            preferred_element_type=jnp.float32)
        dist = (xn + s2) + cn
        m = jnp.min(dist, axis=0, keepdims=True)
        idx = jnp.min(jnp.where(dist == m, kio, k_count), axis=0)
        out_ref[j] = idx.reshape(out_ref.shape[1:])


def kernel(z_e_x, codebook):
    B, D, H, W = z_e_x.shape
    K = codebook.shape[0]
    N = H * W
    z = z_e_x.reshape(B, D, N)
    IMGS = 2
    out = pl.pallas_call(
        _vq_body,
        grid=(B // IMGS,),
        in_specs=[
            pl.BlockSpec((IMGS, D, N), lambda i: (i, 0, 0)),
            pl.BlockSpec((K, D), lambda i: (0, 0)),
        ],
        out_specs=pl.BlockSpec((IMGS, 1, N), lambda i: (i, 0, 0)),
        out_shape=jax.ShapeDtypeStruct((B, 1, N), jnp.int32),
    )(z, codebook)
    return out.reshape(B, H, W)


# speedup vs baseline: 1.9902x; 1.0016x over previous
import jax
import jax.numpy as jnp
from jax.experimental import pallas as pl


def _vq_body(x_ref, cb_ref, out_ref):
    imgs = x_ref.shape[0]
    cb = cb_ref[...]
    k_count, _ = cb.shape
    cb2 = -2.0 * cb
    cn = jnp.sum(cb * cb, axis=1, keepdims=True)
    kio = jax.lax.broadcasted_iota(
        jnp.int32, (k_count, x_ref.shape[2]), 0)
    xs = [x_ref[j] for j in range(imgs)]
    xns = [jnp.sum(x * x, axis=0, keepdims=True) for x in xs]
    s2s = [jax.lax.dot_general(
        cb2, x, (((1,), (0,)), ((), ())),
        preferred_element_type=jnp.float32) for x in xs]
    for j in range(imgs):
        dist = (xns[j] + s2s[j]) + cn
        m = jnp.min(dist, axis=0, keepdims=True)
        idx = jnp.min(jnp.where(dist == m, kio, k_count), axis=0)
        out_ref[j] = idx.reshape(out_ref.shape[1:])


def kernel(z_e_x, codebook):
    B, D, H, W = z_e_x.shape
    K = codebook.shape[0]
    N = H * W
    z = z_e_x.reshape(B, D, N)
    IMGS = 4
    out = pl.pallas_call(
        _vq_body,
        grid=(B // IMGS,),
        in_specs=[
            pl.BlockSpec((IMGS, D, N), lambda i: (i, 0, 0)),
            pl.BlockSpec((K, D), lambda i: (0, 0)),
        ],
        out_specs=pl.BlockSpec((IMGS, 1, N), lambda i: (i, 0, 0)),
        out_shape=jax.ShapeDtypeStruct((B, 1, N), jnp.int32),
    )(z, codebook)
    return out.reshape(B, H, W)


# speedup vs baseline: 2.1309x; 1.0707x over previous
import jax
import jax.numpy as jnp
from jax.experimental import pallas as pl


def _vq_body(x_ref, cb_ref, out_ref):
    imgs = x_ref.shape[0]
    blk = x_ref.shape[2]
    cb = cb_ref[...]
    k_count, _ = cb.shape
    cb2 = -2.0 * cb
    cn = jnp.sum(cb * cb, axis=1, keepdims=True)
    kiof = jax.lax.broadcasted_iota(
        jnp.int32, (k_count, blk), 0).astype(jnp.float32)
    sentinel = jnp.float32(k_count)
    xs = [x_ref[j] for j in range(imgs)]
    xns = [jnp.sum(x * x, axis=0, keepdims=True) for x in xs]
    s2s = [jax.lax.dot_general(
        cb2, x, (((1,), (0,)), ((), ())),
        preferred_element_type=jnp.float32) for x in xs]
    for j in range(imgs):
        dist = (xns[j] + s2s[j]) + cn
        m = jnp.min(dist, axis=0, keepdims=True)
        idx_f = jnp.min(jnp.where(dist == m, kiof, sentinel), axis=0)
        out_ref[j] = idx_f.astype(jnp.int32).reshape(out_ref.shape[1:])


def kernel(z_e_x, codebook):
    B, D, H, W = z_e_x.shape
    K = codebook.shape[0]
    N = H * W
    z = z_e_x.reshape(B, D, N)
    IMGS = 4
    out = pl.pallas_call(
        _vq_body,
        grid=(B // IMGS,),
        in_specs=[
            pl.BlockSpec((IMGS, D, N), lambda i: (i, 0, 0)),
            pl.BlockSpec((K, D), lambda i: (0, 0)),
        ],
        out_specs=pl.BlockSpec((IMGS, 1, N), lambda i: (i, 0, 0)),
        out_shape=jax.ShapeDtypeStruct((B, 1, N), jnp.int32),
    )(z, codebook)
    return out.reshape(B, H, W)
